# Initial kernel scaffold; baseline (speedup 1.0000x reference)
import jax
import jax.numpy as jnp
from jax.experimental import pallas as pl


def kernel(q, queue):
    raise NotImplementedError("write your pallas kernel here")


# speedup vs baseline: 1.0620x; 1.0620x over previous
import numpy as np
import jax
import jax.numpy as jnp
from jax.experimental import pallas as pl
from jax.experimental.pallas import tpu as pltpu

_B = 1024
_DIM = 128
_K = 65536
_T = 0.07
_NH = 1024
_N1 = 256
_N2 = 256
_N3 = 256
_N4 = 64
_EPS = 1e-12


def _build_consts():
    rk = jax.random.key(42)
    k1, k2, k3, k4, k5, k6, k7, k8, k9 = jax.random.split(rk, 9)
    idxs1 = np.asarray(jax.random.randint(k1, (_B, _N1), 0, _NH))
    alpha = np.asarray(jax.random.uniform(k2, (_B, _N1, 1), dtype=jnp.float32))[..., 0] * 0.5
    idxs2 = np.asarray(jax.random.randint(k3, (_B, _N2), 0, _NH))
    beta = 1.0 + np.asarray(jax.random.uniform(k4, (_B, _N2, 1), dtype=jnp.float32))[..., 0] * 0.5
    i3a = np.asarray(jax.random.randint(k5, (_B, _N3), 0, _NH))
    i3b = np.asarray(jax.random.randint(k6, (_B, _N3), 0, _NH))
    gamma = np.asarray(jax.random.uniform(k7, (_B, _N3, 1), dtype=jnp.float32))[..., 0]
    i4 = np.asarray(jax.random.randint(k8, (_B, _N4), 0, _NH))
    noise = np.asarray(jax.random.normal(k9, (_B, _N4, _DIM), dtype=jnp.float32)) * 0.1

    a = alpha.astype(np.float32)
    c1a, c1b = a, 1.0 - a
    d1a, d1b = a * a + (1 - a) ** 2, 2 * a * (1 - a)
    c = (1.0 - beta).astype(np.float32)
    c2a, c2b = c, beta.astype(np.float32)
    d2a, d2b = c * c + beta * beta, 2 * c * beta
    g = gamma.astype(np.float32)
    c3a, c3b = g, 1.0 - g
    d3a, d3b = g * g + (1 - g) ** 2, 2 * g * (1 - g)
    nn2 = (noise * noise).sum(-1).astype(np.float32)
    return dict(
        idxs1=idxs1, idxs2=idxs2, i3a=i3a, i3b=i3b, i4=i4,
        noise=noise.astype(np.float32),
        c1a=c1a.astype(np.float32), c1b=c1b.astype(np.float32),
        d1a=d1a.astype(np.float32), d1b=d1b.astype(np.float32),
        c2a=c2a.astype(np.float32), c2b=c2b.astype(np.float32),
        d2a=d2a.astype(np.float32), d2b=d2b.astype(np.float32),
        c3a=c3a.astype(np.float32), c3b=c3b.astype(np.float32),
        d3a=d3a.astype(np.float32), d3b=d3b.astype(np.float32),
        nn2=nn2,
    )


_C = _build_consts()

_KB = 4096


def _mm_body(q_ref, queue_ref, out_ref):
    qb = q_ref[...]
    nrm = jnp.sqrt(jnp.sum(qb * qb, axis=1, keepdims=True))
    qn = qb / (nrm + _EPS)
    acc = jax.lax.dot_general(
        qn, queue_ref[...], (((1,), (1,)), ((), ())),
        preferred_element_type=jnp.float32)
    out_ref[...] = acc * (1.0 / _T)


def _logits_neg(q, queue):
    return pl.pallas_call(
        _mm_body,
        grid=(_K // _KB,),
        in_specs=[
            pl.BlockSpec((_B, _DIM), lambda j: (0, 0)),
            pl.BlockSpec((_KB, _DIM), lambda j: (j, 0)),
        ],
        out_specs=pl.BlockSpec((_B, _KB), lambda j: (0, j)),
        out_shape=jax.ShapeDtypeStruct((_B, _K), jnp.float32),
    )(q, queue)


def kernel(q, queue):
    lneg_s = _logits_neg(q, queue)
    vals_s, idxs = jax.lax.top_k(lneg_s, _NH)

    nrm = jnp.sqrt(jnp.sum(q * q, axis=1, keepdims=True))
    qn = q / (nrm + _EPS)

    s1 = jnp.take_along_axis(vals_s, jnp.asarray(_C["idxs1"]), axis=1) * _T
    s2 = jnp.take_along_axis(vals_s, jnp.asarray(_C["idxs2"]), axis=1) * _T
    s3a = jnp.take_along_axis(vals_s, jnp.asarray(_C["i3a"]), axis=1) * _T
    s3b = jnp.take_along_axis(vals_s, jnp.asarray(_C["i3b"]), axis=1) * _T
    s4 = jnp.take_along_axis(vals_s, jnp.asarray(_C["i4"]), axis=1) * _T

    g3a = jnp.take_along_axis(idxs, jnp.asarray(_C["i3a"]), axis=1)
    g3b = jnp.take_along_axis(idxs, jnp.asarray(_C["i3b"]), axis=1)
    g4 = jnp.take_along_axis(idxs, jnp.asarray(_C["i4"]), axis=1)

    h3a = queue[g3a]
    h3b = queue[g3b]
    h4 = queue[g4]
    d12 = jnp.einsum("bnd,bnd->bn", h3a, h3b)
    noise = jnp.asarray(_C["noise"])
    dh4 = jnp.einsum("bnd,bnd->bn", h4, noise)
    qn4 = jnp.einsum("bd,bnd->bn", qn, noise)

    lh1 = (_C["c1a"] + _C["c1b"] * s1) / (jnp.sqrt(_C["d1a"] + _C["d1b"] * s1) + _EPS)
    lh2 = (_C["c2a"] + _C["c2b"] * s2) / (jnp.sqrt(_C["d2a"] + _C["d2b"] * s2) + _EPS)
    lh3 = (_C["c3a"] * s3a + _C["c3b"] * s3b) / (
        jnp.sqrt(_C["d3a"] + _C["d3b"] * d12) + _EPS)
    lh4 = (s4 + qn4) / (jnp.sqrt(1.0 + 2.0 * dh4 + _C["nn2"]) + _EPS)

    tail = jnp.concatenate([lh1, lh2, lh3, lh4], axis=1) * (1.0 / _T)
    return jnp.concatenate([lneg_s, tail], axis=1)
